# baseline (speedup 1.0000x reference)
import jax
import jax.numpy as jnp
from jax import lax
from jax.experimental import pallas as pl
from jax.experimental.pallas import tpu as pltpu
from jax.experimental.pallas import tpu_sc as plsc

VOCAB = 1000000
MAXLEN = 200
EMBED = 64
BATCH = 1024

NC, NS = 2, 16
NW = NC * NS
ROWS = BATCH * MAXLEN
CHUNK = 100
CPW = ROWS // (NW * CHUNK)
LANES = 16


def _body(x_ref, tok_ref, pos_ref, out_ref, pos_v, idx_v, tok_v, dsem):
    wid = lax.axis_index("s") * NC + lax.axis_index("c")
    base_chunk = wid * CPW

    pltpu.sync_copy(pos_ref, pos_v)
    pltpu.sync_copy(x_ref.at[pl.ds(base_chunk, CPW)], idx_v)

    def chunk_body(c, carry):
        pltpu.async_copy(tok_ref.at[idx_v.at[c]], tok_v, dsem).wait()
        half = (c % 2) * CHUNK

        def row_body(r, carry2):
            p = half + r
            for d in range(EMBED // LANES):
                sl = pl.ds(d * LANES, LANES)
                tok_v[r, sl] += pos_v[p, sl]
            return carry2

        lax.fori_loop(0, CHUNK, row_body, 0, unroll=2)
        pltpu.sync_copy(tok_v, out_ref.at[base_chunk + c])
        return carry

    lax.fori_loop(0, CPW, chunk_body, 0)


def kernel(x, token_table, pos_table):
    x2 = x.reshape(NW * CPW, CHUNK)
    mesh = plsc.VectorSubcoreMesh(
        core_axis_name="c", subcore_axis_name="s",
        num_cores=NC, num_subcores=NS)
    out = pl.kernel(
        _body,
        out_type=jax.ShapeDtypeStruct((NW * CPW, CHUNK, EMBED), jnp.float32),
        mesh=mesh,
        scratch_types=[
            pltpu.VMEM((MAXLEN, EMBED), jnp.float32),
            pltpu.VMEM((CPW, CHUNK), jnp.int32),
            pltpu.VMEM((CHUNK, EMBED), jnp.float32),
            pltpu.SemaphoreType.DMA,
        ],
        compiler_params=pltpu.CompilerParams(use_tc_tiling_on_sc=False),
    )(x2, token_table, pos_table)
    return out.reshape(BATCH, MAXLEN, EMBED)


# speedup vs baseline: 1.2729x; 1.2729x over previous
import jax
import jax.numpy as jnp
from jax import lax
from jax.experimental import pallas as pl
from jax.experimental.pallas import tpu as pltpu
from jax.experimental.pallas import tpu_sc as plsc

VOCAB = 1000000
MAXLEN = 200
EMBED = 64
BATCH = 1024
PADDED = 128

NC, NS = 2, 16
NW = NC * NS
ROWS = BATCH * MAXLEN
RPW = ROWS // NW
CHUNK = 128
CPW = RPW // CHUNK
LANES = 16


def _body(x_ref, tok_ref, pos_ref, out_ref, pos_v, idx_v, g_v, o_v, g_sem,
          o_sem):
    wid = lax.axis_index("s") * NC + lax.axis_index("c")
    base = wid * RPW

    pltpu.sync_copy(pos_ref, pos_v)
    pltpu.sync_copy(x_ref.at[pl.ds(base, RPW)], idx_v)

    def start_gather(c, buf):
        pltpu.async_copy(
            tok_ref.at[idx_v.at[pl.ds(c * CHUNK, CHUNK)]],
            g_v.at[buf], g_sem.at[buf])

    start_gather(0, 0)
    start_gather(1, 1)

    def chunk_body(c, carry):
        b = c % 2
        pltpu.make_async_copy(
            tok_ref.at[idx_v.at[pl.ds(c * CHUNK, CHUNK)]],
            g_v.at[b], g_sem.at[b]).wait()

        @pl.when(c >= 2)
        def _():
            pltpu.make_async_copy(
                o_v.at[b], out_ref.at[pl.ds(base, CHUNK)], o_sem.at[b]).wait()

        phase = (c * CHUNK) % MAXLEN

        def row_body(r, carry2):
            q = phase + r
            p = q - jnp.where(q >= MAXLEN, MAXLEN, 0)
            for d in range(EMBED // LANES):
                sl = pl.ds(d * LANES, LANES)
                o_v[b, r, sl] = g_v[b, r, sl] + pos_v[p, sl]
            return carry2

        lax.fori_loop(0, CHUNK, row_body, 0, unroll=2)

        pltpu.async_copy(
            o_v.at[b], out_ref.at[pl.ds(base + c * CHUNK, CHUNK)],
            o_sem.at[b])

        @pl.when(c + 2 < CPW)
        def _():
            start_gather(c + 2, b)

        return carry

    lax.fori_loop(0, CPW, chunk_body, 0)

    pltpu.make_async_copy(
        o_v.at[0], out_ref.at[pl.ds(base, CHUNK)], o_sem.at[0]).wait()
    pltpu.make_async_copy(
        o_v.at[1], out_ref.at[pl.ds(base, CHUNK)], o_sem.at[1]).wait()


def kernel(x, token_table, pos_table):
    x1 = x.reshape(ROWS)
    table128 = jnp.pad(token_table, ((0, 0), (0, PADDED - EMBED)))
    mesh = plsc.VectorSubcoreMesh(
        core_axis_name="c", subcore_axis_name="s",
        num_cores=NC, num_subcores=NS)
    out = pl.kernel(
        _body,
        out_type=jax.ShapeDtypeStruct((ROWS, EMBED), jnp.float32),
        mesh=mesh,
        scratch_types=[
            pltpu.VMEM((MAXLEN, EMBED), jnp.float32),
            pltpu.VMEM((RPW,), jnp.int32),
            pltpu.VMEM((2, CHUNK, PADDED), jnp.float32),
            pltpu.VMEM((2, CHUNK, EMBED), jnp.float32),
            pltpu.SemaphoreType.DMA((2,)),
            pltpu.SemaphoreType.DMA((2,)),
        ],
    )(x1, table128, pos_table)
    return out.reshape(BATCH, MAXLEN, EMBED)
